# baseline (speedup 1.0000x reference)
import jax
import jax.numpy as jnp
from jax.experimental import pallas as pl
from jax.experimental.pallas import tpu as pltpu

_N = 131072
_C = 128
_B = 16
_BR = 4096


def _inject_body(oidx_ref, ym_ref, g_ref, wt_ref, pb_ref, out_ref):
    i = pl.program_id(0)
    lo = i * _BR
    out_ref[...] = g_ref[...] + (ym_ref[...] * wt_ref[...] + pb_ref[...])
    for k in range(_B):
        r = oidx_ref[k] - lo

        @pl.when((r >= 0) & (r < _BR))
        def _():
            w = (jnp.clip(r, 0, _BR - 1) // 8) * 8
            rows8 = jax.lax.broadcasted_iota(jnp.int32, (8, 1), 0) + (lo + w)
            z = jnp.ones((8, 1), jnp.float32)
            for j in range(_B):
                z = jnp.where(rows8 == oidx_ref[j], 0.0, z)
            yw = ym_ref[pl.ds(w, 8), :]
            out_ref[pl.ds(w, 8), :] = g_ref[pl.ds(w, 8), :] + z * (
                yw * wt_ref[...] + pb_ref[pl.ds(w, 8), :]
            )


def kernel(y, context_ptr, graph_h, W, b):
    idx = context_ptr[1:].astype(jnp.int32) - 1
    idx = jnp.where(idx < 0, idx + _N, idx)
    oidx = 2 * idx + 1
    ym = jnp.stack([jnp.zeros_like(y), y], axis=1).reshape(2 * _N, 1)
    wt = W.reshape(1, _C)
    parity = jnp.tile(jnp.array([0.0, 1.0], dtype=jnp.float32), _BR // 2)
    pb = parity.reshape(_BR, 1) * b.reshape(1, _C)

    grid_spec = pltpu.PrefetchScalarGridSpec(
        num_scalar_prefetch=1,
        grid=(2 * _N // _BR,),
        in_specs=[
            pl.BlockSpec((_BR, 1), lambda i, oidx_ref: (i, 0)),
            pl.BlockSpec((_BR, _C), lambda i, oidx_ref: (i, 0)),
            pl.BlockSpec((1, _C), lambda i, oidx_ref: (0, 0)),
            pl.BlockSpec((_BR, _C), lambda i, oidx_ref: (0, 0)),
        ],
        out_specs=pl.BlockSpec((_BR, _C), lambda i, oidx_ref: (i, 0)),
    )
    out = pl.pallas_call(
        _inject_body,
        grid_spec=grid_spec,
        out_shape=jax.ShapeDtypeStruct((2 * _N, _C), jnp.float32),
    )(oidx, ym, graph_h, wt, pb)
    return out


# speedup vs baseline: 1.0351x; 1.0351x over previous
import jax
import jax.numpy as jnp
from jax.experimental import pallas as pl
from jax.experimental.pallas import tpu as pltpu

_N = 131072
_C = 128
_B = 16
_CH = 2048
_DEPTH = 8
_NCH = 2 * _N // _CH


def _body(oidx_ref, ym_hbm, wt_ref, pb_ref, g_hbm, out_hbm,
          gbuf, ybuf, obuf, insem, ysem, outsem):
    def start_in(c):
        slot = jax.lax.rem(c, _DEPTH)
        pltpu.make_async_copy(
            g_hbm.at[pl.ds(c * _CH, _CH), :], gbuf.at[slot], insem.at[slot]
        ).start()
        pltpu.make_async_copy(
            ym_hbm.at[pl.ds(c * _CH, _CH), :], ybuf.at[slot], ysem.at[slot]
        ).start()

    def start_out(c):
        slot = jax.lax.rem(c, _DEPTH)
        pltpu.make_async_copy(
            obuf.at[slot], out_hbm.at[pl.ds(c * _CH, _CH), :], outsem.at[slot]
        ).start()

    def wait_in(c):
        slot = jax.lax.rem(c, _DEPTH)
        pltpu.make_async_copy(
            g_hbm.at[pl.ds(c * _CH, _CH), :], gbuf.at[slot], insem.at[slot]
        ).wait()
        pltpu.make_async_copy(
            ym_hbm.at[pl.ds(c * _CH, _CH), :], ybuf.at[slot], ysem.at[slot]
        ).wait()

    def wait_out(c):
        slot = jax.lax.rem(c, _DEPTH)
        pltpu.make_async_copy(
            obuf.at[slot], out_hbm.at[pl.ds(c * _CH, _CH), :], outsem.at[slot]
        ).wait()

    for p in range(_DEPTH):
        start_in(p)

    def loop(c, _):
        slot = jax.lax.rem(c, _DEPTH)
        wait_in(c)

        @pl.when(c >= _DEPTH)
        def _():
            wait_out(c - _DEPTH)

        lo = c * _CH
        obuf[slot] = gbuf[slot] + (ybuf[slot] * wt_ref[...] + pb_ref[...])

        for k in range(_B):
            r = oidx_ref[0, k] - lo

            @pl.when((r >= 0) & (r < _CH))
            def _():
                w = (jnp.clip(r, 0, _CH - 1) // 8) * 8
                rows8 = jax.lax.broadcasted_iota(jnp.int32, (8, 1), 0) + (lo + w)
                z = jnp.ones((8, 1), jnp.float32)
                for j in range(_B):
                    z = jnp.where(rows8 == oidx_ref[0, j], 0.0, z)
                yw = ybuf[slot, pl.ds(w, 8), :]
                obuf[slot, pl.ds(w, 8), :] = gbuf[slot, pl.ds(w, 8), :] + z * (
                    yw * wt_ref[...] + pb_ref[pl.ds(w, 8), :]
                )

        start_out(c)

        @pl.when(c + _DEPTH < _NCH)
        def _():
            start_in(c + _DEPTH)

        return 0

    jax.lax.fori_loop(0, _NCH, loop, 0)

    for p in range(_DEPTH):
        wait_out(_NCH - _DEPTH + p)


def kernel(y, context_ptr, graph_h, W, b):
    idx = context_ptr[1:].astype(jnp.int32) - 1
    idx = jnp.where(idx < 0, idx + _N, idx)
    oidx = (2 * idx + 1).reshape(1, _B)
    ym = jnp.stack([jnp.zeros_like(y), y], axis=1).reshape(2 * _N, 1)
    wt = W.reshape(1, _C)
    parity = jnp.tile(jnp.array([0.0, 1.0], dtype=jnp.float32), _CH // 2)
    pb = parity.reshape(_CH, 1) * b.reshape(1, _C)

    out = pl.pallas_call(
        _body,
        in_specs=[
            pl.BlockSpec(memory_space=pltpu.MemorySpace.SMEM),
            pl.BlockSpec(memory_space=pltpu.MemorySpace.HBM),
            pl.BlockSpec(memory_space=pltpu.MemorySpace.VMEM),
            pl.BlockSpec(memory_space=pltpu.MemorySpace.VMEM),
            pl.BlockSpec(memory_space=pltpu.MemorySpace.HBM),
        ],
        out_specs=pl.BlockSpec(memory_space=pltpu.MemorySpace.HBM),
        out_shape=jax.ShapeDtypeStruct((2 * _N, _C), jnp.float32),
        scratch_shapes=[
            pltpu.VMEM((_DEPTH, _CH, _C), jnp.float32),
            pltpu.VMEM((_DEPTH, _CH, 1), jnp.float32),
            pltpu.VMEM((_DEPTH, _CH, _C), jnp.float32),
            pltpu.SemaphoreType.DMA((_DEPTH,)),
            pltpu.SemaphoreType.DMA((_DEPTH,)),
            pltpu.SemaphoreType.DMA((_DEPTH,)),
        ],
    )(oidx, ym, wt, pb, graph_h)
    return out


# speedup vs baseline: 2.2354x; 2.1596x over previous
import jax
import jax.numpy as jnp
from jax.experimental import pallas as pl
from jax.experimental.pallas import tpu as pltpu

_N = 131072
_C = 128
_B = 16
_CH = 2048
_DEPTH = 8
_RING = 2 * _DEPTH
_NCH = 2 * _N // _CH


def _body(oidx_ref, yfix_ref, y_hbm, wt_ref, b_ref, g_hbm, out_hbm,
          gbuf, ybuf, insem, ysem, outsem):
    def g_in(c):
        slot = jax.lax.rem(c, _RING)
        return pltpu.make_async_copy(
            g_hbm.at[pl.ds(c * _CH, _CH), :], gbuf.at[slot],
            insem.at[jax.lax.rem(c, _DEPTH)])

    def y_in(c):
        slot = jax.lax.rem(c, _RING)
        return pltpu.make_async_copy(
            y_hbm.at[pl.ds(c * (_CH // 2), _CH // 2), :], ybuf.at[slot],
            ysem.at[jax.lax.rem(c, _DEPTH)])

    def g_out(c):
        slot = jax.lax.rem(c, _RING)
        return pltpu.make_async_copy(
            gbuf.at[slot], out_hbm.at[pl.ds(c * _CH, _CH), :],
            outsem.at[jax.lax.rem(c, _DEPTH)])

    for p in range(_DEPTH):
        g_in(p).start()
        y_in(p).start()

    def loop(c, _):
        slot = jax.lax.rem(c, _RING)
        g_in(c).wait()
        y_in(c).wait()

        @pl.when(c >= _DEPTH)
        def _():
            g_out(c - _DEPTH).wait()

        lo = c * _CH
        odd = pl.Slice(1, _CH // 2, 2)
        enc = ybuf[slot] * wt_ref[...] + b_ref[...]
        gbuf[slot, odd, :] = gbuf[slot, odd, :] + enc

        for k in range(_B):
            r = oidx_ref[0, k] - lo

            @pl.when((r >= 0) & (r < _CH))
            def _():
                rc = jnp.clip(r, 0, _CH - 1)
                w = (rc // 8) * 8
                rows8 = jax.lax.broadcasted_iota(jnp.int32, (8, 1), 0) + w
                sel = (rows8 == rc).astype(jnp.float32)
                enc_k = yfix_ref[0, k] * wt_ref[...] + b_ref[...]
                gbuf[slot, pl.ds(w, 8), :] = (
                    gbuf[slot, pl.ds(w, 8), :] - sel * enc_k
                )

        g_out(c).start()

        @pl.when(c + _DEPTH < _NCH)
        def _():
            g_in(c + _DEPTH).start()
            y_in(c + _DEPTH).start()

        return 0

    jax.lax.fori_loop(0, _NCH, loop, 0)

    for p in range(_DEPTH):
        g_out(_NCH - _DEPTH + p).wait()


def kernel(y, context_ptr, graph_h, W, b):
    idx = context_ptr[1:].astype(jnp.int32) - 1
    idx = jnp.where(idx < 0, idx + _N, idx)
    yfix = y[idx, 0].reshape(1, _B)
    oidx = 2 * idx + 1
    order = jnp.argsort(oidx)
    so = oidx[order]
    dup = jnp.concatenate([jnp.zeros((1,), bool), so[1:] == so[:-1]])
    so = jnp.where(dup, -1, so)
    oidx = so.reshape(1, _B)
    yfix = yfix[0, order].reshape(1, _B)
    wt = W.reshape(1, _C)
    b2 = b.reshape(1, _C)

    out = pl.pallas_call(
        _body,
        in_specs=[
            pl.BlockSpec(memory_space=pltpu.MemorySpace.SMEM),
            pl.BlockSpec(memory_space=pltpu.MemorySpace.SMEM),
            pl.BlockSpec(memory_space=pltpu.MemorySpace.HBM),
            pl.BlockSpec(memory_space=pltpu.MemorySpace.VMEM),
            pl.BlockSpec(memory_space=pltpu.MemorySpace.VMEM),
            pl.BlockSpec(memory_space=pltpu.MemorySpace.HBM),
        ],
        out_specs=pl.BlockSpec(memory_space=pltpu.MemorySpace.HBM),
        out_shape=jax.ShapeDtypeStruct((2 * _N, _C), jnp.float32),
        scratch_shapes=[
            pltpu.VMEM((_RING, _CH, _C), jnp.float32),
            pltpu.VMEM((_RING, _CH // 2, 1), jnp.float32),
            pltpu.SemaphoreType.DMA((_DEPTH,)),
            pltpu.SemaphoreType.DMA((_DEPTH,)),
            pltpu.SemaphoreType.DMA((_DEPTH,)),
        ],
    )(oidx, yfix, y, wt, b2, graph_h)
    return out
